# baseline (speedup 1.0000x reference)
import functools

import jax
import jax.numpy as jnp
from jax import lax
from jax.experimental import pallas as pl
from jax.experimental.pallas import tpu as pltpu
from jax.experimental.pallas import tpu_sc as plsc

_N = 10000
_D = 128
_E = 320000
_NSC = 2
_NSUB = 16
_NW = _NSC * _NSUB
_CS = 64
_CH = 160
_NPH = 4
_HCH = _CH // _NPH
_PW = _CH * _CS
_EPAD = _NW * _PW
_AGG_ROWS = 10240
_ZROWS = _AGG_ROWS // _NSUB


def _sc_agg_body(h_hbm, src_hbm, dst_hbm, zeros_hbm, out_hbm,
                 src_v, dst_v, gb0, gb1, gb2, gb3, agg_sh,
                 gsem0, gsem1, gsem2, gsem3, ssem0, ssem1, ssem2, ssem3):
    c = lax.axis_index("c")
    s = lax.axis_index("s")
    wid = c * _NSUB + s
    pltpu.sync_copy(zeros_hbm, agg_sh.at[pl.ds(s * _ZROWS, _ZROWS)])
    plsc.subcore_barrier()

    gbufs = (gb0, gb1, gb2, gb3)
    gsems = (gsem0, gsem1, gsem2, gsem3)
    ssems = (ssem0, ssem1, ssem2, ssem3)

    def wait_gather(j, b):
        pltpu.make_async_copy(h_hbm.at[src_v.at[j]], gbufs[b],
                              gsems[b]).wait()

    def issue_gather(j, b):
        pltpu.async_copy(h_hbm.at[src_v.at[j]], gbufs[b], gsems[b])

    def issue_scatter(j, b):
        pltpu.async_copy(gbufs[b], agg_sh.at[dst_v.at[j]], ssems[b],
                         add=True)

    def wait_scatter(b):
        pltpu.make_async_copy(gbufs[b], agg_sh.at[dst_v.at[0]],
                              ssems[b]).wait()

    for phase in range(_NPH):
        pltpu.sync_copy(src_hbm.at[wid, pl.ds(phase * _HCH, _HCH)], src_v)
        pltpu.sync_copy(dst_hbm.at[wid, pl.ds(phase * _HCH, _HCH)], dst_v)
        issue_gather(0, 0)
        issue_gather(1, 1)
        for k in range(4):
            wait_gather(k, k)
            issue_scatter(k, k)
            if k >= 2:
                wait_scatter(k - 2)
            issue_gather(k + 2, (k + 2) % 4)

        def body(kk, carry):
            for b in range(4):
                k = 4 * kk + b
                wait_gather(k, b)
                issue_scatter(k, b)
                m = k + 2
                bm = (b + 2) % 4

                @pl.when(m < _HCH)
                def _():
                    wait_scatter(bm)
                    issue_gather(m, bm)
            return carry

        lax.fori_loop(1, _HCH // 4, body, 0)
        for b in range(4):
            wait_scatter(b)
    plsc.subcore_barrier()
    pltpu.sync_copy(agg_sh.at[pl.ds(s * _ZROWS, _ZROWS)],
                    out_hbm.at[c, pl.ds(s * _ZROWS, _ZROWS)])


@jax.jit
def _sc_agg(h, src_p, dst_p, zeros):
    k = pl.kernel(
        _sc_agg_body,
        out_type=jax.ShapeDtypeStruct((_NSC, _AGG_ROWS, _D), jnp.float32),
        mesh=plsc.VectorSubcoreMesh(core_axis_name="c", subcore_axis_name="s"),
        scratch_types=(
            [pltpu.VMEM((_HCH, _CS), jnp.int32)] * 2
            + [pltpu.VMEM((_CS, _D), jnp.float32)] * 4
            + [pltpu.VMEM_SHARED((_AGG_ROWS, _D), jnp.float32)]
            + [pltpu.SemaphoreType.DMA] * 8
        ),
    )
    return k(h, src_p, dst_p, zeros)


def _bn_body(x_ref, imp_ref, g_ref, b_ref, o_ref):
    y = x_ref[...] * imp_ref[...]
    m = jnp.mean(y, axis=0, keepdims=True)
    d = y - m
    v = jnp.mean(d * d, axis=0, keepdims=True)
    o_ref[...] = d * lax.rsqrt(v + 1e-5) * g_ref[...] + b_ref[...]


def _bn(x, imp, g, b):
    return pl.pallas_call(
        _bn_body,
        out_shape=jax.ShapeDtypeStruct((_N, _D), jnp.float32),
    )(x, imp, g.reshape(1, _D), b.reshape(1, _D))


def _mlp_body(h_ref, a_ref, w_ref, b_ref, o_ref):
    x = h_ref[...] + a_ref[0, :_N] + a_ref[1, :_N]
    o_ref[...] = jnp.tanh(
        jnp.dot(x, w_ref[...], preferred_element_type=jnp.float32)
        + b_ref[...])


def _mlp(h, aggp, w, b):
    return pl.pallas_call(
        _mlp_body,
        out_shape=jax.ShapeDtypeStruct((_N, _D), jnp.float32),
    )(h, aggp, w, b.reshape(1, _D))


def _mlp_fc_body(h_ref, a_ref, w_ref, b_ref, wfc_ref, o5_ref, o6_ref):
    x = h_ref[...] + a_ref[0, :_N] + a_ref[1, :_N]
    h5 = jnp.tanh(
        jnp.dot(x, w_ref[...], preferred_element_type=jnp.float32)
        + b_ref[...])
    o5_ref[...] = h5
    o6_ref[...] = jnp.tanh(
        jnp.dot(h5, wfc_ref[...], preferred_element_type=jnp.float32))


def _mlp_fc(h, aggp, w, b, wfc):
    return pl.pallas_call(
        _mlp_fc_body,
        out_shape=(jax.ShapeDtypeStruct((_N, _D), jnp.float32),
                   jax.ShapeDtypeStruct((_N, _D), jnp.float32)),
    )(h, aggp, w, b.reshape(1, _D), wfc)


def _gin(X, imp, ei, g, b, Ws, bs, Wfc, zeros):
    pad = _EPAD - _E
    ar = jnp.arange(pad, dtype=jnp.int32)
    src_p = jnp.concatenate([ei[0], ar % _N]).reshape(_NW, _CH, _CS)
    dst_p = jnp.concatenate(
        [ei[1], _N + (ar % (_AGG_ROWS - _N))]).reshape(_NW, _CH, _CS)
    h = _bn(X, imp, g, b)
    hs = []
    for i, (W, bb) in enumerate(zip(Ws, bs)):
        aggp = _sc_agg(h, src_p, dst_p, zeros)
        if i < 4:
            h = _mlp(h, aggp, W, bb)
            hs.append(h)
        else:
            h5, h6 = _mlp_fc(h, aggp, W, bb, Wfc)
            hs.append(h5)
            hs.append(h6)
    return jnp.concatenate(hs, axis=-1)


def kernel(source_x, source_x_importance, source_edge_index, target_x,
           target_x_importance, target_edge_index, bn_gamma, bn_beta,
           W1, b1, W2, b2, W3, b3, W4, b4, W5, b5, Wfc):
    Ws = [W1, W2, W3, W4, W5]
    bs = [b1, b2, b3, b4, b5]
    zeros = jnp.zeros((_ZROWS, _D), jnp.float32)
    out_s = _gin(source_x, source_x_importance, source_edge_index,
                 bn_gamma, bn_beta, Ws, bs, Wfc, zeros)
    out_t = _gin(target_x, target_x_importance, target_edge_index,
                 bn_gamma, bn_beta, Ws, bs, Wfc, zeros)
    return (out_s, out_t)


# speedup vs baseline: 1.2240x; 1.2240x over previous
import functools

import jax
import jax.numpy as jnp
from jax import lax
from jax.experimental import pallas as pl
from jax.experimental.pallas import tpu as pltpu
from jax.experimental.pallas import tpu_sc as plsc

_N = 10000
_D = 128
_E = 320000
_NSC = 2
_NSUB = 16
_CS = 128
_CH = 160
_NPH = 4
_HCH = _CH // _NPH
_PW = _CH * _CS
_EPAD = _NSUB * _PW
_AGG_ROWS = 10240
_ZROWS = _AGG_ROWS // _NSUB


def _sc_agg_body(h_hbm, src_hbm, dst_hbm, zeros_hbm, out_hbm,
                 src_v, dst_v, gb0, gb1, agg_sh, gsem0, gsem1):
    c = lax.axis_index("c")
    s = lax.axis_index("s")
    gbufs = (gb0, gb1)
    gsems = (gsem0, gsem1)

    def load_idx(phase):
        pltpu.sync_copy(src_hbm.at[c, s, pl.ds(phase * _HCH, _HCH)], src_v)
        pltpu.sync_copy(dst_hbm.at[c, s, pl.ds(phase * _HCH, _HCH)], dst_v)

    def issue_gather(j, b):
        pltpu.async_copy(h_hbm.at[c].at[src_v.at[j]], gbufs[b], gsems[b])

    def wait_gather(j, b):
        pltpu.make_async_copy(h_hbm.at[c].at[src_v.at[j]], gbufs[b],
                              gsems[b]).wait()

    load_idx(0)
    issue_gather(0, 0)
    issue_gather(1, 1)
    pltpu.sync_copy(zeros_hbm.at[pl.ds(s * _ZROWS, _ZROWS)],
                    agg_sh.at[pl.ds(s * _ZROWS, _ZROWS)])
    plsc.subcore_barrier()

    for phase in range(_NPH):
        if phase > 0:
            load_idx(phase)
            issue_gather(0, 0)
            issue_gather(1, 1)

        def body(jj, carry):
            for b in range(2):
                j = 2 * jj + b
                wait_gather(j, b)
                pltpu.sync_copy(gbufs[b], agg_sh.at[dst_v.at[j]], add=True)
                nxt = j + 2

                @pl.when(nxt < _HCH)
                def _():
                    issue_gather(nxt, b)
            return carry

        lax.fori_loop(0, _HCH // 2, body, 0)

    plsc.subcore_barrier()
    pltpu.sync_copy(agg_sh.at[pl.ds(s * _ZROWS, _ZROWS)],
                    out_hbm.at[c, pl.ds(s * _ZROWS, _ZROWS)])


@jax.jit
def _sc_agg(h_stack, src_p, dst_p, zeros):
    k = pl.kernel(
        _sc_agg_body,
        out_type=jax.ShapeDtypeStruct((_NSC, _AGG_ROWS, _D), jnp.float32),
        mesh=plsc.VectorSubcoreMesh(core_axis_name="c", subcore_axis_name="s"),
        scratch_types=(
            [pltpu.VMEM((_HCH, _CS), jnp.int32)] * 2
            + [pltpu.VMEM((_CS, _D), jnp.float32)] * 2
            + [pltpu.VMEM_SHARED((_AGG_ROWS, _D), jnp.float32)]
            + [pltpu.SemaphoreType.DMA] * 2
        ),
    )
    return k(h_stack, src_p, dst_p, zeros)


def _bn_body(x_ref, imp_ref, g_ref, b_ref, o_ref):
    y = x_ref[...] * imp_ref[...]
    m = jnp.mean(y, axis=1, keepdims=True)
    d = y - m
    v = jnp.mean(d * d, axis=1, keepdims=True)
    o_ref[...] = d * lax.rsqrt(v + 1e-5) * g_ref[...] + b_ref[...]


def _bn(x, imp, g, b):
    return pl.pallas_call(
        _bn_body,
        out_shape=jax.ShapeDtypeStruct((_NSC, _N, _D), jnp.float32),
    )(x, imp, g.reshape(1, 1, _D), b.reshape(1, 1, _D))


def _mlp_body(h_ref, a_ref, w_ref, b_ref, o_ref):
    x = (h_ref[...] + a_ref[:, :_N]).reshape(_NSC * _N, _D)
    o_ref[...] = jnp.tanh(
        jnp.dot(x, w_ref[...], preferred_element_type=jnp.float32)
        + b_ref[...]).reshape(_NSC, _N, _D)


def _mlp(h, agg, w, b):
    return pl.pallas_call(
        _mlp_body,
        out_shape=jax.ShapeDtypeStruct((_NSC, _N, _D), jnp.float32),
    )(h, agg, w, b.reshape(1, _D))


def _mlp_fc_body(h_ref, a_ref, w_ref, b_ref, wfc_ref, o5_ref, o6_ref):
    x = (h_ref[...] + a_ref[:, :_N]).reshape(_NSC * _N, _D)
    h5 = jnp.tanh(
        jnp.dot(x, w_ref[...], preferred_element_type=jnp.float32)
        + b_ref[...])
    o5_ref[...] = h5.reshape(_NSC, _N, _D)
    o6_ref[...] = jnp.tanh(
        jnp.dot(h5, wfc_ref[...],
                preferred_element_type=jnp.float32)).reshape(_NSC, _N, _D)


def _mlp_fc(h, agg, w, b, wfc):
    return pl.pallas_call(
        _mlp_fc_body,
        out_shape=(jax.ShapeDtypeStruct((_NSC, _N, _D), jnp.float32),
                   jax.ShapeDtypeStruct((_NSC, _N, _D), jnp.float32)),
    )(h, agg, w, b.reshape(1, _D), wfc)


def _pad_edges(ei):
    pad = _EPAD - _E
    ar = jnp.arange(pad, dtype=jnp.int32)
    src = jnp.concatenate([ei[0], ar % _N]).reshape(_NSUB, _CH, _CS)
    dst = jnp.concatenate(
        [ei[1], _N + (ar % (_AGG_ROWS - _N))]).reshape(_NSUB, _CH, _CS)
    return src, dst


def kernel(source_x, source_x_importance, source_edge_index, target_x,
           target_x_importance, target_edge_index, bn_gamma, bn_beta,
           W1, b1, W2, b2, W3, b3, W4, b4, W5, b5, Wfc):
    Ws = [W1, W2, W3, W4, W5]
    bs = [b1, b2, b3, b4, b5]
    zeros = jnp.zeros((_AGG_ROWS, _D), jnp.float32)
    ss, sd = _pad_edges(source_edge_index)
    ts, td = _pad_edges(target_edge_index)
    src_p = jnp.stack([ss, ts])
    dst_p = jnp.stack([sd, td])
    x = jnp.stack([source_x, target_x])
    imp = jnp.stack([source_x_importance, target_x_importance])

    h = _bn(x, imp, bn_gamma, bn_beta)
    hs = []
    for i, (W, bb) in enumerate(zip(Ws, bs)):
        agg = _sc_agg(h, src_p, dst_p, zeros)
        if i < 4:
            h = _mlp(h, agg, W, bb)
            hs.append(h)
        else:
            h5, h6 = _mlp_fc(h, agg, W, bb, Wfc)
            hs.append(h5)
            hs.append(h6)
    out_s = jnp.concatenate([hh[0] for hh in hs], axis=-1)
    out_t = jnp.concatenate([hh[1] for hh in hs], axis=-1)
    return (out_s, out_t)
